# baseline (speedup 1.0000x reference)
import functools

import jax
import jax.numpy as jnp
from jax import lax
from jax.experimental import pallas as pl
from jax.experimental.pallas import tpu as pltpu
from jax.experimental.pallas import tpu_sc as plsc

_VOCAB = 1000000
_DIM = 64
_OUT_DIM = 5
_SEQ = 200
_BATCH = 4096

_NC = 2
_NS = 16
_NW = _NC * _NS
_BPW = _BATCH // _NW
_LANES = 16
_PDIM = 2 * _DIM
_ICH = _BPW // _LANES
_DEPTH = 4


def _sc_pool_body(text_hbm, emb2_hbm, oute_hbm, outo_hbm,
                  idx_v, idxe_v, idxo_v, poole_v, poolo_v, sem_e, sem_o):
    wid = lax.axis_index("s") * _NC + lax.axis_index("c")
    base = wid * _BPW

    pltpu.sync_copy(text_hbm.at[:, pl.ds(base, _BPW)], idx_v)

    one = jnp.full((_LANES,), 1, jnp.int32)
    zero_i = jnp.zeros((_LANES,), jnp.int32)
    zero_f = jnp.zeros((_LANES,), jnp.float32)

    def split_body(s, carry):
        for c in range(_ICH):
            sl = pl.ds(c * _LANES, _LANES)
            v = idx_v[s, sl]
            p = lax.shift_right_logical(v, one)
            odd = lax.bitwise_and(v, one) == one
            idxe_v[s, sl] = jnp.where(odd, zero_i, p)
            idxo_v[s, sl] = jnp.where(odd, p, zero_i)
        return carry

    lax.fori_loop(0, _SEQ, split_body, 0)

    def zbody(i, carry):
        for c in range(_PDIM // _LANES):
            sl = pl.ds(c * _LANES, _LANES)
            poole_v[i, sl] = zero_f
            poolo_v[i, sl] = zero_f
        return carry

    lax.fori_loop(0, _BPW, zbody, 0)

    def fire(idx_ref, s, pool, sem):
        pltpu.async_copy(emb2_hbm.at[idx_ref.at[s]], pool, sem, add=True)

    def drain(idx_ref, pool, sem):
        pltpu.make_async_copy(emb2_hbm.at[idx_ref.at[0]], pool, sem).wait()

    for s in range(_DEPTH):
        fire(idxe_v, s, poole_v, sem_e)
        fire(idxo_v, s, poolo_v, sem_o)

    def body(p, carry):
        drain(idxe_v, poole_v, sem_e)

        @pl.when(p + _DEPTH < _SEQ)
        def _():
            fire(idxe_v, p + _DEPTH, poole_v, sem_e)

        drain(idxo_v, poolo_v, sem_o)

        @pl.when(p + _DEPTH < _SEQ)
        def _():
            fire(idxo_v, p + _DEPTH, poolo_v, sem_o)

        return carry

    lax.fori_loop(0, _SEQ, body, 0)

    pltpu.sync_copy(poole_v, oute_hbm.at[pl.ds(base, _BPW), :])
    pltpu.sync_copy(poolo_v, outo_hbm.at[pl.ds(base, _BPW), :])


@functools.partial(
    pl.kernel,
    out_type=[
        jax.ShapeDtypeStruct((_BATCH, _PDIM), jnp.float32),
        jax.ShapeDtypeStruct((_BATCH, _PDIM), jnp.float32),
    ],
    mesh=plsc.VectorSubcoreMesh(core_axis_name="c", subcore_axis_name="s"),
    compiler_params=pltpu.CompilerParams(use_tc_tiling_on_sc=True),
    scratch_types=[
        pltpu.VMEM((_SEQ, _BPW), jnp.int32),
        pltpu.VMEM((_SEQ, _BPW), jnp.int32),
        pltpu.VMEM((_SEQ, _BPW), jnp.int32),
        pltpu.VMEM((_BPW, _PDIM), jnp.float32),
        pltpu.VMEM((_BPW, _PDIM), jnp.float32),
        pltpu.SemaphoreType.DMA,
        pltpu.SemaphoreType.DMA,
    ],
)
def _sc_pool(text_hbm, emb2_hbm, oute_hbm, outo_hbm,
             idx_v, idxe_v, idxo_v, poole_v, poolo_v, sem_e, sem_o):
    _sc_pool_body(text_hbm, emb2_hbm, oute_hbm, outo_hbm,
                  idx_v, idxe_v, idxo_v, poole_v, poolo_v, sem_e, sem_o)


def _finish_body(pe_ref, po_ref, text_ref, emb01_ref, w_ref, b_ref, o_ref):
    pooled = pe_ref[:, :_DIM] + po_ref[:, _DIM:]
    n_odd = jnp.sum(
        lax.bitwise_and(text_ref[...], 1), axis=0, dtype=jnp.int32
    ).astype(jnp.float32)[:, None]
    n_even = float(_SEQ) - n_odd
    pooled = pooled - n_odd * emb01_ref[0:1, :] - n_even * emb01_ref[1:2, :]
    acc = lax.dot_general(pooled, w_ref[...], (((1,), (1,)), ((), ())),
                          preferred_element_type=jnp.float32)
    o_ref[...] = acc * (1.0 / _SEQ) + b_ref[...]


def kernel(text, emb, W, b):
    text = text.astype(jnp.int32)
    emb2 = emb.reshape(_VOCAB // 2, _PDIM)
    sums_e, sums_o = _sc_pool(text, emb2)
    out = pl.pallas_call(
        _finish_body,
        out_shape=jax.ShapeDtypeStruct((_BATCH, _OUT_DIM), jnp.float32),
    )(sums_e, sums_o, text, lax.slice(emb, (0, 0), (2, _DIM)), W,
      b.reshape(1, _OUT_DIM))
    return out


# speedup vs baseline: 28.6497x; 28.6497x over previous
import functools

import jax
import jax.numpy as jnp
from jax import lax
from jax.experimental import pallas as pl
from jax.experimental.pallas import tpu as pltpu
from jax.experimental.pallas import tpu_sc as plsc

_VOCAB = 1000000
_DIM = 64
_OUT_DIM = 5
_SEQ = 200
_BATCH = 4096

_NC = 2
_NS = 16
_NW = _NC * _NS
_BPW = _BATCH // _NW
_LANES = 16
_PW = 8
_VBLK = 5000
_DEPTH = 8


def _project_body(emb_ref, w_ref, p_ref):
    w = w_ref[...] * (1.0 / _SEQ)
    p_ref[...] = lax.dot_general(emb_ref[...], w, (((1,), (1,)), ((), ())),
                                 preferred_element_type=jnp.float32)


def _project(emb, w8):
    return pl.pallas_call(
        _project_body,
        grid=(_VOCAB // _VBLK,),
        in_specs=[
            pl.BlockSpec((_VBLK, _DIM), lambda i: (i, 0)),
            pl.BlockSpec((_PW, _DIM), lambda i: (0, 0)),
        ],
        out_specs=pl.BlockSpec((_VBLK, _PW), lambda i: (i, 0)),
        out_shape=jax.ShapeDtypeStruct((_VOCAB, _PW), jnp.float32),
    )(emb, w8)


def _sc_pool_body(text_hbm, p_hbm, out_hbm, idx_v, pool_v, sem):
    wid = lax.axis_index("s") * _NC + lax.axis_index("c")
    base = wid * _BPW

    pltpu.sync_copy(text_hbm.at[:, pl.ds(base, _BPW)], idx_v)

    def fire(s, add=True):
        pltpu.async_copy(p_hbm.at[idx_v.at[s]], pool_v, sem, add=add)

    def drain():
        pltpu.make_async_copy(p_hbm.at[idx_v.at[0]], pool_v, sem).wait()

    fire(0, add=False)
    drain()
    for s in range(1, _DEPTH + 1):
        fire(s)

    def body(p, carry):
        drain()

        @pl.when(p + _DEPTH + 1 < _SEQ)
        def _():
            fire(p + _DEPTH + 1)

        return carry

    lax.fori_loop(0, _SEQ - 1, body, 0)

    pltpu.sync_copy(pool_v, out_hbm.at[pl.ds(base, _BPW), :])


@functools.partial(
    pl.kernel,
    out_type=jax.ShapeDtypeStruct((_BATCH, _PW), jnp.float32),
    mesh=plsc.VectorSubcoreMesh(core_axis_name="c", subcore_axis_name="s"),
    compiler_params=pltpu.CompilerParams(use_tc_tiling_on_sc=False),
    scratch_types=[
        pltpu.VMEM((_SEQ, _BPW), jnp.int32),
        pltpu.VMEM((_BPW, _PW), jnp.float32),
        pltpu.SemaphoreType.DMA,
    ],
)
def _sc_pool(text_hbm, p_hbm, out_hbm, idx_v, pool_v, sem):
    _sc_pool_body(text_hbm, p_hbm, out_hbm, idx_v, pool_v, sem)


def _finish_body(p_ref, b_ref, o_ref):
    o_ref[...] = p_ref[:, :_OUT_DIM] + b_ref[...]


def kernel(text, emb, W, b):
    text = text.astype(jnp.int32)
    w8 = jnp.zeros((_PW, _DIM), jnp.float32).at[:_OUT_DIM].set(W)
    proj = _project(emb, w8)
    sums = _sc_pool(text, proj)
    out = pl.pallas_call(
        _finish_body,
        out_shape=jax.ShapeDtypeStruct((_BATCH, _OUT_DIM), jnp.float32),
    )(sums, b.reshape(1, _OUT_DIM))
    return out


# speedup vs baseline: 93.7328x; 3.2717x over previous
import functools

import jax
import jax.numpy as jnp
from jax import lax
from jax.experimental import pallas as pl
from jax.experimental.pallas import tpu as pltpu
from jax.experimental.pallas import tpu_sc as plsc

_VOCAB = 1000000
_DIM = 64
_OUT_DIM = 5
_SEQ = 200
_BATCH = 4096

_NC = 2
_NS = 16
_NW = _NC * _NS
_BPW = _BATCH // _NW
_LANES = 16
_PW = 8
_VBLK = 8192
_DEPTH = 8


def _project_body(embt_ref, w_ref, p_ref):
    w = w_ref[...] * (1.0 / _SEQ)
    mm = lax.dot_general(embt_ref[...], w, (((0,), (1,)), ((), ())),
                         preferred_element_type=jnp.float32)
    p_ref[:, 0:_PW] = mm


def _project(embt, w8):
    grid = (_VOCAB + _VBLK - 1) // _VBLK
    return pl.pallas_call(
        _project_body,
        grid=(grid,),
        in_specs=[
            pl.BlockSpec((_DIM, _VBLK), lambda i: (0, i)),
            pl.BlockSpec((_PW, _DIM), lambda i: (0, 0)),
        ],
        out_specs=pl.BlockSpec((_VBLK, 128), lambda i: (i, 0)),
        out_shape=jax.ShapeDtypeStruct((_VOCAB, 128), jnp.float32),
    )(embt, w8)


def _sc_pool_body(text_hbm, p_hbm, out_hbm, idx_v, idx8_v, pool_v, sem):
    wid = lax.axis_index("s") * _NC + lax.axis_index("c")
    base = wid * _BPW

    pltpu.sync_copy(text_hbm.at[:, pl.ds(base, _BPW)], idx_v)

    three = jnp.full((_LANES,), 3, jnp.int32)

    def shift_body(s, carry):
        for c in range(_BPW // _LANES):
            sl = pl.ds(c * _LANES, _LANES)
            idx8_v[s, sl] = lax.shift_left(idx_v[s, sl], three)
        return carry

    lax.fori_loop(0, _SEQ, shift_body, 0)

    def fire(s, add=True):
        pltpu.async_copy(p_hbm.at[idx8_v.at[s]], pool_v, sem, add=add)

    def drain():
        pltpu.make_async_copy(p_hbm.at[idx8_v.at[0]], pool_v, sem).wait()

    fire(0, add=False)
    drain()
    for s in range(1, _DEPTH + 1):
        fire(s)

    def body(p, carry):
        drain()

        @pl.when(p + _DEPTH + 1 < _SEQ)
        def _():
            fire(p + _DEPTH + 1)

        return carry

    lax.fori_loop(0, _SEQ - 1, body, 0)

    pltpu.sync_copy(pool_v, out_hbm.at[pl.ds(base, _BPW), :])


@functools.partial(
    pl.kernel,
    out_type=jax.ShapeDtypeStruct((_BATCH, 2 * _PW), jnp.float32),
    mesh=plsc.VectorSubcoreMesh(core_axis_name="c", subcore_axis_name="s"),
    compiler_params=pltpu.CompilerParams(use_tc_tiling_on_sc=False),
    scratch_types=[
        pltpu.VMEM((_SEQ, _BPW), jnp.int32),
        pltpu.VMEM((_SEQ, _BPW), jnp.int32),
        pltpu.VMEM((_BPW, 2 * _PW), jnp.float32),
        pltpu.SemaphoreType.DMA,
    ],
)
def _sc_pool(text_hbm, p_hbm, out_hbm, idx_v, idx8_v, pool_v, sem):
    _sc_pool_body(text_hbm, p_hbm, out_hbm, idx_v, idx8_v, pool_v, sem)


def _finish_body(p_ref, b_ref, o_ref):
    o_ref[...] = p_ref[:, :_OUT_DIM] + b_ref[...]


def kernel(text, emb, W, b):
    text = text.astype(jnp.int32)
    w8 = jnp.zeros((_PW, _DIM), jnp.float32).at[:_OUT_DIM].set(W)
    proj = _project(emb.T, w8)
    sums = _sc_pool(text, proj.reshape(8 * _VOCAB, 2 * _PW))
    out = pl.pallas_call(
        _finish_body,
        out_shape=jax.ShapeDtypeStruct((_BATCH, _OUT_DIM), jnp.float32),
    )(sums, b.reshape(1, _OUT_DIM))
    return out
